# baseline (speedup 1.0000x reference)
import jax
import jax.numpy as jnp
from jax import lax
from jax.experimental import pallas as pl
from jax.experimental.pallas import tpu as pltpu
from jax.experimental.pallas import tpu_sc as plsc

_B = 4096
_T = 200
_D = 32
_V = 1000000
_N = _B * _T
_NC = 2
_NS = 16
_NW = _NC * _NS
_RPW = _N // _NW
_CHUNK = 320
_NCHUNK = _RPW // _CHUNK
_G = 80
_NG = _CHUNK // _G
_PACK = 128 // _D
_OCHUNK = _CHUNK // _PACK
_LANES = 16
_NBLK = _CHUNK // _LANES


def _body(xrow_hbm, xsub_hbm, tab_hbm, pos_hbm, out_hbm,
          idxr_a, idxr_b, idxs_a, idxs_b, rows_a, rows_b,
          comp_a, comp_b, pos_v, sg_a, sg_b, swb_a, swb_b):
    wid = lax.axis_index("s") * _NC + lax.axis_index("c")
    base = wid * _RPW
    obase = wid * (_RPW // _PACK)

    pltpu.sync_copy(pos_hbm, pos_v)

    idxr = (idxr_a, idxr_b)
    idxs = (idxs_a, idxs_b)
    rows = (rows_a, rows_b)
    comp = (comp_a, comp_b)
    sg = (sg_a, sg_b)
    swb = (swb_a, swb_b)

    def fire(c, p):
        off = base + c * _CHUNK
        pltpu.sync_copy(xrow_hbm.at[pl.ds(off, _CHUNK)], idxr[p])
        pltpu.sync_copy(xsub_hbm.at[pl.ds(off, _CHUNK)], idxs[p])
        for g in range(_NG):
            pltpu.async_copy(
                tab_hbm.at[idxr[p].at[pl.ds(g * _G, _G)]],
                rows[p].at[pl.ds(g * _G, _G)],
                sg[p])

    def drain(p):
        for g in range(_NG):
            pltpu.make_async_copy(
                tab_hbm.at[idxr[p].at[pl.ds(g * _G, _G)]],
                rows[p].at[pl.ds(g * _G, _G)],
                sg[p]).wait()

    def compact(c, p):
        rv = rows[p]
        cv = comp[p]
        t0 = lax.rem(c * _CHUNK, _T)

        def bbody(b, carry):
            r_vec = b * _LANES + lax.iota(jnp.int32, _LANES)
            sub_vec = idxs[p][pl.ds(b * _LANES, _LANES)]
            t_vec = lax.rem(t0 + r_vec, _T)
            tq = lax.shift_right_logical(t_vec, 2)
            subp = lax.shift_left(lax.bitwise_and(t_vec, 3), 5)
            rq = lax.shift_right_logical(r_vec, 2)
            cb = lax.shift_left(lax.bitwise_and(r_vec, 3), 5)
            for cc in range(_D):
                v = plsc.load_gather(rv, [r_vec, sub_vec + cc])
                pvv = plsc.load_gather(pos_v, [tq, subp + cc])
                plsc.store_scatter(cv, [rq, cb + cc], v + pvv)
            return carry

        lax.fori_loop(0, _NBLK, bbody, 0)

    def wb_fire(c, p):
        pltpu.async_copy(
            comp[p], out_hbm.at[pl.ds(obase + c * _OCHUNK, _OCHUNK)], swb[p])

    def wb_wait(c, p):
        pltpu.make_async_copy(
            comp[p], out_hbm.at[pl.ds(obase + c * _OCHUNK, _OCHUNK)],
            swb[p]).wait()

    fire(0, 0)

    def outer(i, carry):
        c0 = 2 * i
        fire(c0 + 1, 1)
        drain(0)

        @pl.when(c0 > 0)
        def _():
            wb_wait(c0 - 2, 0)

        compact(c0, 0)
        wb_fire(c0, 0)

        fire(jnp.minimum(c0 + 2, _NCHUNK - 1), 0)
        drain(1)

        @pl.when(c0 > 0)
        def _():
            wb_wait(c0 - 1, 1)

        compact(c0 + 1, 1)
        wb_fire(c0 + 1, 1)
        return carry

    lax.fori_loop(0, _NCHUNK // 2, outer, 0)
    drain(0)
    wb_wait(_NCHUNK - 2, 0)
    wb_wait(_NCHUNK - 1, 1)


def kernel(x, token_table, pos_table):
    xf = x.astype(jnp.int32).reshape(_N)
    xrow = lax.shift_right_logical(xf, 2)
    xsub = lax.shift_left(lax.bitwise_and(xf, 3), 5)
    tab128 = token_table.reshape(_V // _PACK, 128)
    pos128 = pos_table.reshape(_T // _PACK, 128)
    mesh = plsc.VectorSubcoreMesh(core_axis_name="c", subcore_axis_name="s")
    out128 = pl.kernel(
        _body,
        out_type=jax.ShapeDtypeStruct((_N // _PACK, 128), jnp.float32),
        mesh=mesh,
        compiler_params=pltpu.CompilerParams(needs_layout_passes=False),
        scratch_types=[
            pltpu.VMEM((_CHUNK,), jnp.int32),
            pltpu.VMEM((_CHUNK,), jnp.int32),
            pltpu.VMEM((_CHUNK,), jnp.int32),
            pltpu.VMEM((_CHUNK,), jnp.int32),
            pltpu.VMEM((_CHUNK, 128), jnp.float32),
            pltpu.VMEM((_CHUNK, 128), jnp.float32),
            pltpu.VMEM((_OCHUNK, 128), jnp.float32),
            pltpu.VMEM((_OCHUNK, 128), jnp.float32),
            pltpu.VMEM((_T // _PACK, 128), jnp.float32),
            pltpu.SemaphoreType.DMA,
            pltpu.SemaphoreType.DMA,
            pltpu.SemaphoreType.DMA,
            pltpu.SemaphoreType.DMA,
        ],
    )(xrow, xsub, tab128, pos128)
    return out128.reshape(_B, _T, _D)


# speedup vs baseline: 2.0247x; 2.0247x over previous
import jax
import jax.numpy as jnp
from jax import lax
from jax.experimental import pallas as pl
from jax.experimental.pallas import tpu as pltpu
from jax.experimental.pallas import tpu_sc as plsc

_B = 4096
_T = 200
_D = 32
_V = 1000000
_N = _B * _T
_NC = 2
_NS = 16
_NW = _NC * _NS
_LANES = 16
_TO = 8
_NU = _T // _TO
_ROWS = _TO * 128
_R = _D // 8


def _body(xt_hbm, tab_hbm, pos_hbm, out_hbm,
          xv, rows_v, slab_v, pos_v, sg, swb):
    wid = lax.axis_index("s") * _NC + lax.axis_index("c")
    c = wid

    pltpu.sync_copy(pos_hbm, pos_v)

    def gather_desc(ti):
        return pltpu.make_async_copy(
            tab_hbm.at[xv.at[ti]],
            rows_v.at[pl.ds(ti * 128, 128)],
            sg)

    def unit(tt, carry):
        pltpu.sync_copy(
            xt_hbm.at[pl.ds(tt * _TO, _TO), pl.ds(c * 128, 128)], xv)
        for ti in range(_TO):
            pltpu.async_copy(
                tab_hbm.at[xv.at[ti]],
                rows_v.at[pl.ds(ti * 128, 128)],
                sg)

        def tbody(ti, carry2):
            gather_desc(ti).wait()
            t = tt * _TO + ti
            par = lax.bitwise_and(ti, 1)
            @pl.when(t >= 2)
            def _():
                for r in range(_R):
                    pltpu.make_async_copy(
                        slab_v.at[par, r],
                        out_hbm.at[t, r, c],
                        swb).wait()
            bt = jnp.broadcast_to(t, (_LANES,))
            lrows = [
                ti * 128 + lg * _LANES + lax.iota(jnp.int32, _LANES)
                for lg in range(8)
            ]
            for r in range(_R):
                for s in range(8):
                    d = 8 * r + s
                    bc = jnp.full((_LANES,), d, jnp.int32)
                    pv = plsc.load_gather(pos_v, [bt, bc])
                    for lg in range(8):
                        v = plsc.load_gather(rows_v, [lrows[lg], bc])
                        slab_v[par, r, s, pl.ds(lg * _LANES, _LANES)] = v + pv
            for r in range(_R):
                pltpu.async_copy(slab_v.at[par, r], out_hbm.at[t, r, c], swb)
            return carry2

        lax.fori_loop(0, _TO, tbody, 0)
        return carry

    lax.fori_loop(0, _NU, unit, 0)
    for par, toff in ((0, 2), (1, 1)):
        t = _T - toff
        for r in range(_R):
            pltpu.make_async_copy(
                slab_v.at[par, r], out_hbm.at[t, r, c], swb).wait()


def kernel(x, token_table, pos_table):
    xt = x.astype(jnp.int32).T
    mesh = plsc.VectorSubcoreMesh(core_axis_name="c", subcore_axis_name="s")
    l5 = pl.kernel(
        _body,
        out_type=jax.ShapeDtypeStruct((_T, _R, _NW, 8, 128), jnp.float32),
        mesh=mesh,
        compiler_params=pltpu.CompilerParams(
            use_tc_tiling_on_sc=False, needs_layout_passes=False),
        scratch_types=[
            pltpu.VMEM((_TO, 128), jnp.int32),
            pltpu.VMEM((_ROWS, _D), jnp.float32),
            pltpu.VMEM((2, _R, 8, 128), jnp.float32),
            pltpu.VMEM((_T, _D), jnp.float32),
            pltpu.SemaphoreType.DMA,
            pltpu.SemaphoreType.DMA,
        ],
    )(xt, token_table, pos_table)
    return l5.transpose((2, 4, 0, 1, 3)).reshape(_B, _T, _D)
